# baseline (speedup 1.0000x reference)
import functools

import jax
import jax.numpy as jnp
from jax import lax
from jax.experimental import pallas as pl
from jax.experimental.pallas import tpu as pltpu
from jax.experimental.pallas import tpu_sc as plsc

N = 10000
D = 64
SCALES = (8, 16, 32)
KMAX = 33
R = 128
NPAD = 10112
NBLK = NPAD // R
PAD_COORD = 1e9


def _two_sum(a, b):
    s = a + b
    bp = s - a
    e = (a - (s - bp)) + (b - bp)
    return s, e


def _topk_body(prow_ref, pcol_ref, idx_ref, d2_ref):
    xr = prow_ref[:, 0:1]
    yr = prow_ref[:, 1:2]
    zr = prow_ref[:, 2:3]
    xc = pcol_ref[0:1, :]
    yc = pcol_ref[1:2, :]
    zc = pcol_ref[2:3, :]
    xr2 = prow_ref[:, 3:4]
    yr2 = prow_ref[:, 4:5]
    zr2 = prow_ref[:, 5:6]
    xc2 = pcol_ref[3:4, :]
    yc2 = pcol_ref[4:5, :]
    zc2 = pcol_ref[5:6, :]
    sqr = (xr2 * xr2 + zr2 * zr2) + yr2 * yr2
    sqc = (xc2 * xc2 + zc2 * zc2) + yc2 * yc2
    xx = xr * xc
    yy = yr * yc
    zz = zr * zc
    s1, e1 = _two_sum(xx, yy)
    s2, e2 = _two_sum(s1, zz)
    dot = s2 + (e1 + e2)
    d2 = jnp.maximum((sqr + sqc) - 2.0 * dot, 0.0)
    d2_ref[:] = jnp.sqrt(d2)

    col_iota = lax.broadcasted_iota(jnp.int32, (R, NPAD), 1)
    lane64 = lax.broadcasted_iota(jnp.int32, (R, 64), 1)

    def step(k, carry):
        prev_idx, acc = carry
        d2c = d2_ref[:]
        d2n = jnp.where(col_iota == prev_idx, jnp.float32(jnp.inf), d2c)
        d2_ref[:] = d2n
        m = jnp.min(d2n, axis=1, keepdims=True)
        idx = jnp.min(
            jnp.where(d2n == m, col_iota, jnp.int32(NPAD)),
            axis=1, keepdims=True)
        acc = jnp.where(lane64 == k, idx, acc)
        return idx, acc

    init = (jnp.full((R, 1), -1, jnp.int32), jnp.zeros((R, 64), jnp.int32))
    _, acc = lax.fori_loop(0, KMAX, step, init)
    idx_ref[:] = acc


def _bf16_rne(v):
    u = lax.bitcast_convert_type(v, jnp.uint32)
    r = (u + jnp.uint32(0x7FFF) + ((u >> 16) & jnp.uint32(1))) \
        & jnp.uint32(0xFFFF0000)
    return lax.bitcast_convert_type(r, jnp.float32)


def _topk_call(points, interpret=False):
    pr = _bf16_rne(points)
    p_rows = jnp.zeros((NPAD, 8), jnp.float32)
    p_rows = p_rows.at[:, :6].set(PAD_COORD)
    p_rows = p_rows.at[:N, :3].set(pr).at[:N, 3:6].set(points)
    p_cols = jnp.zeros((8, NPAD), jnp.float32)
    p_cols = p_cols.at[:6, :].set(PAD_COORD)
    p_cols = p_cols.at[:3, :N].set(pr.T).at[3:6, :N].set(points.T)
    return pl.pallas_call(
        _topk_body,
        grid=(NBLK,),
        in_specs=[
            pl.BlockSpec((R, 8), lambda i: (i, 0)),
            pl.BlockSpec((8, NPAD), lambda i: (0, 0)),
        ],
        out_specs=pl.BlockSpec((R, 64), lambda i: (i, 0)),
        out_shape=jax.ShapeDtypeStruct((NPAD, 64), jnp.int32),
        scratch_shapes=[pltpu.VMEM((R, NPAD), jnp.float32)],
        interpret=interpret,
    )(p_rows, p_cols)


_CH = 448


@functools.lru_cache(maxsize=None)
def _make_gather(B, nchunks, nw):
    mesh = plsc.VectorSubcoreMesh(core_axis_name="c", subcore_axis_name="s")
    bpw = B // nw

    npairs = nchunks // 2
    tail = nchunks % 2

    @functools.partial(
        pl.kernel, mesh=mesh,
        out_type=jax.ShapeDtypeStruct((B, 128), jnp.float32),
        scratch_types=[
            pltpu.VMEM((_CH,), jnp.int32),
            pltpu.VMEM((_CH,), jnp.int32),
            pltpu.VMEM((_CH, 128), jnp.float32),
            pltpu.VMEM((_CH, 128), jnp.float32),
            pltpu.SemaphoreType.DMA,
            pltpu.SemaphoreType.DMA,
            pltpu.SemaphoreType.DMA,
            pltpu.SemaphoreType.DMA,
        ],
    )
    def gather(table_hbm, idx_hbm, out_hbm, ia, ib, ra, rb,
               gsa, gsb, wsa, wsb):
        ncores = plsc.get_sparse_core_info().num_cores
        wid = lax.axis_index("s") * ncores + lax.axis_index("c")
        base = wid * bpw

        def issue(iv, rv, gs, c):
            pltpu.sync_copy(idx_hbm.at[pl.ds(base + c * _CH, _CH)], iv)
            pltpu.async_copy(table_hbm.at[iv], rv, gs)

        def wait_g(iv, rv, gs):
            pltpu.make_async_copy(table_hbm.at[iv], rv, gs).wait()

        def put(rv, ws, c):
            pltpu.async_copy(rv, out_hbm.at[pl.ds(base + c * _CH, _CH)], ws)

        def wait_w(rv, ws, c):
            pltpu.make_async_copy(
                rv, out_hbm.at[pl.ds(base + c * _CH, _CH)], ws).wait()

        issue(ia, ra, gsa, jnp.int32(0))

        def body(g, carry):
            c0 = g * 2
            c1 = c0 + 1

            @pl.when(g >= 1)
            def _():
                wait_w(rb, wsb, c1 - 2)
            issue(ib, rb, gsb, c1)
            wait_g(ia, ra, gsa)
            put(ra, wsa, c0)
            if tail:
                wait_w(ra, wsa, c0)
                issue(ia, ra, gsa, c0 + 2)
            else:
                @pl.when(g + 1 < npairs)
                def _():
                    wait_w(ra, wsa, c0)
                    issue(ia, ra, gsa, c0 + 2)
            wait_g(ib, rb, gsb)
            put(rb, wsb, c1)
            return carry

        lax.fori_loop(0, npairs, body, 0)
        if tail:
            ctail = nchunks - 1
            wait_g(ia, ra, gsa)
            put(ra, wsa, ctail)
            wait_w(ra, wsa, ctail)
        else:
            wait_w(ra, wsa, nchunks - 2)
        wait_w(rb, wsb, nchunks - 1 - tail)

    return gather


def kernel(feats_c_norm, point_c_norm, f_masks):
    del f_masks
    nw = plsc.get_sparse_core_info().num_cores * \
        plsc.get_sparse_core_info().num_subcores
    idx_all = _topk_call(point_c_norm)
    nbr = idx_all[:N, 1:KMAX]
    table = jnp.zeros((N, 128), jnp.float32).at[:, :D].set(feats_c_norm)
    outs = [feats_c_norm]
    for s in SCALES:
        b = N * s
        step = nw * _CH
        bpad = ((b + step - 1) // step) * step
        flat = jnp.zeros((bpad,), jnp.int32).at[:b].set(nbr[:, :s].reshape(-1))
        g = _make_gather(bpad, bpad // step, nw)(table, flat)
        outs.append(g[:b, :D].reshape(N, s * D))
    return tuple(outs)


# speedup vs baseline: 1.1205x; 1.1205x over previous
import functools

import jax
import jax.numpy as jnp
from jax import lax
from jax.experimental import pallas as pl
from jax.experimental.pallas import tpu as pltpu
from jax.experimental.pallas import tpu_sc as plsc

N = 10000
D = 64
SCALES = (8, 16, 32)
KMAX = 33
R = 128
NPAD = 10112
NBLK = NPAD // R
PAD_COORD = 1e9


def _two_sum(a, b):
    s = a + b
    bp = s - a
    e = (a - (s - bp)) + (b - bp)
    return s, e


def _topk_body(prow_ref, pcol_ref, idx_ref, d2_ref):
    xr = prow_ref[:, 0:1]
    yr = prow_ref[:, 1:2]
    zr = prow_ref[:, 2:3]
    xc = pcol_ref[0:1, :]
    yc = pcol_ref[1:2, :]
    zc = pcol_ref[2:3, :]
    xr2 = prow_ref[:, 3:4]
    yr2 = prow_ref[:, 4:5]
    zr2 = prow_ref[:, 5:6]
    xc2 = pcol_ref[3:4, :]
    yc2 = pcol_ref[4:5, :]
    zc2 = pcol_ref[5:6, :]
    sqr = (xr2 * xr2 + zr2 * zr2) + yr2 * yr2
    sqc = (xc2 * xc2 + zc2 * zc2) + yc2 * yc2
    xx = xr * xc
    yy = yr * yc
    zz = zr * zc
    s1, e1 = _two_sum(xx, yy)
    s2, e2 = _two_sum(s1, zz)
    dot = s2 + (e1 + e2)
    d2 = jnp.maximum((sqr + sqc) - 2.0 * dot, 0.0)
    d2_ref[:] = jnp.sqrt(d2)

    col_iota = lax.broadcasted_iota(jnp.int32, (R, NPAD), 1)
    lane64 = lax.broadcasted_iota(jnp.int32, (R, 64), 1)

    def step(k, acc):
        d2c = d2_ref[:]
        m = jnp.min(d2c, axis=1, keepdims=True)
        idx = jnp.min(
            jnp.where(d2c == m, col_iota, jnp.int32(NPAD)),
            axis=1, keepdims=True)
        acc = jnp.where(lane64 == k, idx, acc)
        d2_ref[:] = jnp.where(col_iota == idx, jnp.float32(jnp.inf), d2c)
        return acc

    idx_ref[:] = lax.fori_loop(0, KMAX, step, jnp.zeros((R, 64), jnp.int32))


def _bf16_rne(v):
    u = lax.bitcast_convert_type(v, jnp.uint32)
    r = (u + jnp.uint32(0x7FFF) + ((u >> 16) & jnp.uint32(1))) \
        & jnp.uint32(0xFFFF0000)
    return lax.bitcast_convert_type(r, jnp.float32)


def _topk_call(points, interpret=False):
    pr = _bf16_rne(points)
    p_rows = jnp.zeros((NPAD, 8), jnp.float32)
    p_rows = p_rows.at[:, :6].set(PAD_COORD)
    p_rows = p_rows.at[:N, :3].set(pr).at[:N, 3:6].set(points)
    p_cols = jnp.zeros((8, NPAD), jnp.float32)
    p_cols = p_cols.at[:6, :].set(PAD_COORD)
    p_cols = p_cols.at[:3, :N].set(pr.T).at[3:6, :N].set(points.T)
    return pl.pallas_call(
        _topk_body,
        grid=(NBLK,),
        in_specs=[
            pl.BlockSpec((R, 8), lambda i: (i, 0)),
            pl.BlockSpec((8, NPAD), lambda i: (0, 0)),
        ],
        out_specs=pl.BlockSpec((R, 64), lambda i: (i, 0)),
        out_shape=jax.ShapeDtypeStruct((NPAD, 64), jnp.int32),
        scratch_shapes=[pltpu.VMEM((R, NPAD), jnp.float32)],
        interpret=interpret,
    )(p_rows, p_cols)


_CH = 512


@functools.lru_cache(maxsize=None)
def _make_gather(B, nchunks, nw):
    mesh = plsc.VectorSubcoreMesh(core_axis_name="c", subcore_axis_name="s")
    bpw = B // nw

    @functools.partial(
        pl.kernel, mesh=mesh,
        out_type=jax.ShapeDtypeStruct((B, 128), jnp.float32),
        scratch_types=[
            pltpu.VMEM((_CH,), jnp.int32),
            pltpu.VMEM((_CH, 128), jnp.float32),
            pltpu.SemaphoreType.DMA,
        ],
    )
    def gather(table_hbm, idx_hbm, out_hbm, idx_v, rows_v, sem):
        ncores = plsc.get_sparse_core_info().num_cores
        wid = lax.axis_index("s") * ncores + lax.axis_index("c")
        base = wid * bpw

        def body(c, carry):
            off = base + c * _CH
            pltpu.sync_copy(idx_hbm.at[pl.ds(off, _CH)], idx_v)
            pltpu.async_copy(table_hbm.at[idx_v], rows_v, sem).wait()
            pltpu.sync_copy(rows_v, out_hbm.at[pl.ds(off, _CH)])
            return carry

        lax.fori_loop(0, nchunks, body, 0)

    return gather


def kernel(feats_c_norm, point_c_norm, f_masks):
    del f_masks
    nw = plsc.get_sparse_core_info().num_cores * \
        plsc.get_sparse_core_info().num_subcores
    idx_all = _topk_call(point_c_norm)
    nbr = idx_all[:N, 1:KMAX]
    table = jnp.zeros((N, 128), jnp.float32).at[:, :D].set(feats_c_norm)
    outs = [feats_c_norm]
    for s in SCALES:
        b = N * s
        step = nw * _CH
        bpad = ((b + step - 1) // step) * step
        flat = jnp.zeros((bpad,), jnp.int32).at[:b].set(nbr[:, :s].reshape(-1))
        g = _make_gather(bpad, bpad // step, nw)(table, flat)
        outs.append(g[:b, :D].reshape(N, s * D))
    return tuple(outs)
